# Initial kernel scaffold; baseline (speedup 1.0000x reference)
import jax
import jax.numpy as jnp
from jax.experimental import pallas as pl


def kernel(x, edge_index, edge_relation, W1, b1, W2, b2):
    raise NotImplementedError("write your pallas kernel here")


# speedup vs baseline: 49.9608x; 49.9608x over previous
import functools

import jax
import jax.numpy as jnp
from jax import lax
from jax.experimental import pallas as pl
from jax.experimental.pallas import tpu as pltpu
from jax.experimental.pallas import tpu_sc as plsc

N = 10000
E = 320000
NREL = 4
DF = 128
DC = 16
NC, NS, L = 2, 16, 16
NW = NC * NS
EW = E // NW
NP = 10240
RPT = NP // NS
BN = 2000
NB = N // BN

_mesh = plsc.VectorSubcoreMesh(core_axis_name="c", subcore_axis_name="s",
                               num_cores=NC, num_subcores=NS)
_sc_params = pltpu.CompilerParams(needs_layout_passes=False,
                                  use_tc_tiling_on_sc=False)


def _deg_gidx_body(ei_hbm, rel_hbm, degp_hbm, gidx_hbm, rowf_hbm,
                   row_v, col_v, rel_v, gidx_v, deg_v):
  c = lax.axis_index("c")
  s = lax.axis_index("s")
  wid = s * NC + c
  base = wid * EW
  pltpu.sync_copy(ei_hbm.at[0, pl.ds(base, EW)], row_v)
  pltpu.sync_copy(ei_hbm.at[1, pl.ds(base, EW)], col_v)
  pltpu.sync_copy(rel_hbm.at[pl.ds(base, EW)], rel_v)

  zeros = jnp.zeros((L,), jnp.float32)

  @pl.loop(0, NP // L)
  def _zero(i):
    deg_v[pl.ds(i * L, L)] = zeros

  ones = jnp.ones((L,), jnp.float32)

  @pl.loop(0, EW // L)
  def _edges(i):
    sl = pl.ds(i * L, L)
    gidx_v[sl] = rel_v[sl] * N + col_v[sl]
    plsc.addupdate_scatter(deg_v, [row_v[sl]], ones)

  pltpu.sync_copy(deg_v, degp_hbm.at[wid])
  pltpu.sync_copy(gidx_v, gidx_hbm.at[pl.ds(base, EW)])
  pltpu.sync_copy(row_v, rowf_hbm.at[pl.ds(base, EW)])


_deg_gidx = pl.kernel(
    _deg_gidx_body,
    out_type=[jax.ShapeDtypeStruct((NW, NP), jnp.float32),
              jax.ShapeDtypeStruct((E,), jnp.int32),
              jax.ShapeDtypeStruct((E,), jnp.int32)],
    mesh=_mesh,
    scratch_types=[pltpu.VMEM((EW,), jnp.int32),
                   pltpu.VMEM((EW,), jnp.int32),
                   pltpu.VMEM((EW,), jnp.int32),
                   pltpu.VMEM((EW,), jnp.int32),
                   pltpu.VMEM((NP,), jnp.float32)],
    compiler_params=_sc_params,
)


def _scatter_body(D, C, G, NG, NSLOT, RB, table_hbm, gidx_hbm, ridx_hbm,
                  out_hbm, *scr):
  ivs = scr[0:2]
  rvs = scr[2:4]
  bufs = scr[4:4 + NSLOT]
  acc_sh = scr[4 + NSLOT]
  isems = scr[5 + NSLOT:7 + NSLOT]
  gsems = scr[7 + NSLOT:7 + 2 * NSLOT]
  ssems = scr[7 + 2 * NSLOT:7 + 3 * NSLOT]
  c = lax.axis_index("c")
  s = lax.axis_index("s")
  wid = s * NC + c
  wbase = wid * EW
  GC = G * C

  zeros = jnp.zeros((L,), jnp.float32)

  @pl.loop(0, RB)
  def _zr(r):
    @pl.loop(0, D // L)
    def _zc(k):
      bufs[0][r, pl.ds(k * L, L)] = zeros

  @pl.loop(0, RPT // RB)
  def _zs(j):
    pltpu.sync_copy(bufs[0].at[pl.ds(0, RB)],
                    acc_sh.at[pl.ds(s * RPT + j * RB, RB)])

  plsc.subcore_barrier()

  def idx_load(g, p):
    pltpu.async_copy(gidx_hbm.at[pl.ds(wbase + g * GC, GC)], ivs[p],
                     isems[p])
    pltpu.async_copy(ridx_hbm.at[pl.ds(wbase + g * GC, GC)], rvs[p],
                     isems[p])

  def idx_wait(g, p):
    pltpu.make_async_copy(gidx_hbm.at[pl.ds(wbase + g * GC, GC)], ivs[p],
                          isems[p]).wait()
    pltpu.make_async_copy(ridx_hbm.at[pl.ds(wbase + g * GC, GC)], rvs[p],
                          isems[p]).wait()

  def gstart(j, k, p):
    pltpu.async_copy(table_hbm.at[ivs[p].at[pl.ds(j * C, C)]], bufs[k],
                     gsems[k])

  def gwait(j, k, p):
    pltpu.make_async_copy(table_hbm.at[ivs[p].at[pl.ds(j * C, C)]],
                          bufs[k], gsems[k]).wait()

  def sstart(j, k, p):
    pltpu.async_copy(bufs[k], acc_sh.at[rvs[p].at[pl.ds(j * C, C)]],
                     ssems[k], add=True)

  def sdrain(k, p):
    pltpu.make_async_copy(bufs[k], acc_sh.at[rvs[p].at[pl.ds(0, C)]],
                          ssems[k]).wait()

  idx_load(0, 0)
  for g in range(NG):
    p = g % 2
    idx_wait(g, p)
    for k in range(NSLOT):
      if g > 0:
        sdrain(k, p)
      gstart(k, k, p)
    if g + 1 < NG:
      idx_load(g + 1, (g + 1) % 2)

    @pl.loop(0, G // NSLOT)
    def _pipe(t):
      for k in range(NSLOT):
        j = t * NSLOT + k
        gwait(j, k, p)
        sstart(j, k, p)

        @pl.when(j + NSLOT < G)
        def _():
          sdrain(k, p)
          gstart(j + NSLOT, k, p)

  for k in range(NSLOT):
    sdrain(k, (NG - 1) % 2)

  plsc.subcore_barrier()

  @pl.loop(0, RPT // RB)
  def _rd(j):
    rs = s * RPT + j * RB
    pltpu.sync_copy(acc_sh.at[pl.ds(rs, RB)], bufs[0].at[pl.ds(0, RB)])
    pltpu.sync_copy(bufs[0].at[pl.ds(0, RB)], out_hbm.at[c, pl.ds(rs, RB)])


def _make_scatter(D, C, G, NG, NSLOT, RB):
  scratch = ([pltpu.VMEM((G * C,), jnp.int32) for _ in range(4)]
             + [pltpu.VMEM((C, D), jnp.float32) for _ in range(NSLOT)]
             + [pltpu.MemorySpace.VMEM_SHARED((NP, D), jnp.float32)]
             + [pltpu.SemaphoreType.DMA] * (2 + 2 * NSLOT))
  return pl.kernel(
      functools.partial(_scatter_body, D, C, G, NG, NSLOT, RB),
      out_type=[jax.ShapeDtypeStruct((NC, NP, D), jnp.float32)],
      mesh=_mesh,
      scratch_types=scratch,
      compiler_params=_sc_params,
  )


_scatter128 = _make_scatter(DF, 40, 25, 10, 5, 40)
_scatter16 = _make_scatter(DC, 80, 25, 5, 5, 80)


def _dinv(dp_ref):
  deg = jnp.sum(dp_ref[...], axis=1)
  return jnp.where(deg > 0, lax.rsqrt(deg), 0.0)


def _tc1_body(dp_ref, x_ref, w1_ref, y_ref):
  xs = (x_ref[...] * _dinv(dp_ref)[:, None]).astype(jnp.bfloat16)
  y_ref[...] = jnp.dot(xs, w1_ref[...], preferred_element_type=jnp.float32)


def _tc2_body(dp_ref, acc_ref, b1_ref, w2_ref, y_ref):
  dinv = _dinv(dp_ref)
  h1 = b1_ref[...] + dinv[:, None] * (acc_ref[0] + acc_ref[1])
  h1s = (h1 * dinv[:, None]).astype(jnp.bfloat16)
  y_ref[...] = jnp.dot(h1s, w2_ref[...], preferred_element_type=jnp.float32)


def _tc3_body(dp_ref, acc_ref, b2_ref, o_ref):
  dinv = _dinv(dp_ref)
  logits = b2_ref[...] + dinv[:, None] * (acc_ref[0] + acc_ref[1])[:, :8]
  m = jnp.max(logits, axis=1, keepdims=True)
  z = jnp.exp(logits - m)
  o_ref[...] = logits - m - jnp.log(jnp.sum(z, axis=1, keepdims=True))


def _tc1(dp, x, w1b):
  return pl.pallas_call(
      _tc1_body,
      grid=(NB, NREL),
      in_specs=[pl.BlockSpec((BN, NW), lambda i, r: (i, 0)),
                pl.BlockSpec((BN, DF), lambda i, r: (i, 0)),
                pl.BlockSpec((DF, DF), lambda i, r: (r, 0))],
      out_specs=pl.BlockSpec((BN, DF), lambda i, r: (r * NB + i, 0)),
      out_shape=jax.ShapeDtypeStruct((NREL * N, DF), jnp.float32),
  )(dp, x, w1b)


def _tc2(dp, acc, b1, w2pb):
  return pl.pallas_call(
      _tc2_body,
      grid=(NB, NREL),
      in_specs=[pl.BlockSpec((BN, NW), lambda i, r: (i, 0)),
                pl.BlockSpec((NC, BN, DF), lambda i, r: (0, i, 0)),
                pl.BlockSpec((1, DF), lambda i, r: (0, 0)),
                pl.BlockSpec((DF, DC), lambda i, r: (r, 0))],
      out_specs=pl.BlockSpec((BN, DC), lambda i, r: (r * NB + i, 0)),
      out_shape=jax.ShapeDtypeStruct((NREL * N, DC), jnp.float32),
  )(dp, acc, b1, w2pb)


def _tc3(dp, acc, b2):
  return pl.pallas_call(
      _tc3_body,
      grid=(NB,),
      in_specs=[pl.BlockSpec((BN, NW), lambda i: (i, 0)),
                pl.BlockSpec((NC, BN, DC), lambda i: (0, i, 0)),
                pl.BlockSpec((1, 8), lambda i: (0, 0))],
      out_specs=pl.BlockSpec((BN, 8), lambda i: (i, 0)),
      out_shape=jax.ShapeDtypeStruct((N, 8), jnp.float32),
  )(dp, acc, b2)


@jax.jit
def kernel(x, edge_index, edge_relation, W1, b1, W2, b2):
  degp, gidx, rowf = _deg_gidx(edge_index, edge_relation)
  dp = degp.T

  y1 = _tc1(dp, x, W1.astype(jnp.bfloat16))
  (acc1,) = _scatter128(y1, gidx, rowf)

  w2pb = jnp.pad(W2, ((0, 0), (0, DC - W2.shape[1]))).astype(jnp.bfloat16)
  y2 = _tc2(dp, acc1, b1.reshape(1, DF), w2pb)
  (acc2,) = _scatter16(y2, gidx, rowf)

  return _tc3(dp, acc2, b2.reshape(1, 8))


# speedup vs baseline: 53.8034x; 1.0769x over previous
import functools

import jax
import jax.numpy as jnp
from jax import lax
from jax.experimental import pallas as pl
from jax.experimental.pallas import tpu as pltpu
from jax.experimental.pallas import tpu_sc as plsc

N = 10000
E = 320000
NREL = 4
DF = 128
DC = 16
NC, NS, L = 2, 16, 16
NW = NC * NS
EW = E // NW
NP = 10240
RPT = NP // NS
BN = 2000
NB = N // BN

_mesh = plsc.VectorSubcoreMesh(core_axis_name="c", subcore_axis_name="s",
                               num_cores=NC, num_subcores=NS)
_sc_params = pltpu.CompilerParams(needs_layout_passes=False,
                                  use_tc_tiling_on_sc=False)


def _deg_gidx_body(ei_hbm, rel_hbm, degp_hbm, gidx_hbm, rowf_hbm,
                   row_v, col_v, rel_v, gidx_v, deg_v):
  c = lax.axis_index("c")
  s = lax.axis_index("s")
  wid = s * NC + c
  base = wid * EW
  pltpu.sync_copy(ei_hbm.at[0, pl.ds(base, EW)], row_v)
  pltpu.sync_copy(ei_hbm.at[1, pl.ds(base, EW)], col_v)
  pltpu.sync_copy(rel_hbm.at[pl.ds(base, EW)], rel_v)

  zeros = jnp.zeros((L,), jnp.float32)

  @pl.loop(0, NP // L)
  def _zero(i):
    deg_v[pl.ds(i * L, L)] = zeros

  ones = jnp.ones((L,), jnp.float32)

  @pl.loop(0, EW // L)
  def _edges(i):
    sl = pl.ds(i * L, L)
    gidx_v[sl] = rel_v[sl] * N + col_v[sl]
    plsc.addupdate_scatter(deg_v, [row_v[sl]], ones)

  pltpu.sync_copy(deg_v, degp_hbm.at[wid])
  pltpu.sync_copy(gidx_v, gidx_hbm.at[pl.ds(base, EW)])
  pltpu.sync_copy(row_v, rowf_hbm.at[pl.ds(base, EW)])


_deg_gidx = pl.kernel(
    _deg_gidx_body,
    out_type=[jax.ShapeDtypeStruct((NW, NP), jnp.float32),
              jax.ShapeDtypeStruct((E,), jnp.int32),
              jax.ShapeDtypeStruct((E,), jnp.int32)],
    mesh=_mesh,
    scratch_types=[pltpu.VMEM((EW,), jnp.int32),
                   pltpu.VMEM((EW,), jnp.int32),
                   pltpu.VMEM((EW,), jnp.int32),
                   pltpu.VMEM((EW,), jnp.int32),
                   pltpu.VMEM((NP,), jnp.float32)],
    compiler_params=_sc_params,
)


def _scatter_body(D, C, G, NG, NSLOT, RB, table_hbm, gidx_hbm, ridx_hbm,
                  out_hbm, *scr):
  ivs = scr[0:2]
  rvs = scr[2:4]
  bufs = scr[4:4 + NSLOT]
  acc_sh = scr[4 + NSLOT]
  isems = scr[5 + NSLOT:7 + NSLOT]
  gsems = scr[7 + NSLOT:7 + 2 * NSLOT]
  ssems = scr[7 + 2 * NSLOT:7 + 3 * NSLOT]
  c = lax.axis_index("c")
  s = lax.axis_index("s")
  wid = s * NC + c
  wbase = wid * EW
  GC = G * C

  zeros = jnp.zeros((L,), jnp.float32)

  @pl.loop(0, RB)
  def _zr(r):
    @pl.loop(0, D // L)
    def _zc(k):
      bufs[0][r, pl.ds(k * L, L)] = zeros

  @pl.loop(0, RPT // RB)
  def _zs(j):
    pltpu.sync_copy(bufs[0].at[pl.ds(0, RB)],
                    acc_sh.at[pl.ds(s * RPT + j * RB, RB)])

  plsc.subcore_barrier()

  def idx_load(g, p):
    pltpu.async_copy(gidx_hbm.at[pl.ds(wbase + g * GC, GC)], ivs[p],
                     isems[p])
    pltpu.async_copy(ridx_hbm.at[pl.ds(wbase + g * GC, GC)], rvs[p],
                     isems[p])

  def idx_wait(g, p):
    pltpu.make_async_copy(gidx_hbm.at[pl.ds(wbase + g * GC, GC)], ivs[p],
                          isems[p]).wait()
    pltpu.make_async_copy(ridx_hbm.at[pl.ds(wbase + g * GC, GC)], rvs[p],
                          isems[p]).wait()

  def gstart(j, k, p):
    pltpu.async_copy(table_hbm.at[ivs[p].at[pl.ds(j * C, C)]], bufs[k],
                     gsems[k])

  def gwait(j, k, p):
    pltpu.make_async_copy(table_hbm.at[ivs[p].at[pl.ds(j * C, C)]],
                          bufs[k], gsems[k]).wait()

  def sstart(j, k, p):
    pltpu.async_copy(bufs[k], acc_sh.at[rvs[p].at[pl.ds(j * C, C)]],
                     ssems[k], add=True)

  def sdrain(k, p):
    pltpu.make_async_copy(bufs[k], acc_sh.at[rvs[p].at[pl.ds(0, C)]],
                          ssems[k]).wait()

  idx_load(0, 0)
  for g in range(NG):
    p = g % 2
    idx_wait(g, p)
    for k in range(NSLOT):
      if g > 0:
        sdrain(k, p)
      gstart(k, k, p)
    if g + 1 < NG:
      idx_load(g + 1, (g + 1) % 2)

    @pl.loop(0, G // NSLOT)
    def _pipe(t):
      for k in range(NSLOT):
        j = t * NSLOT + k
        gwait(j, k, p)
        sstart(j, k, p)

        @pl.when(j + NSLOT < G)
        def _():
          sdrain(k, p)
          gstart(j + NSLOT, k, p)

  for k in range(NSLOT):
    sdrain(k, (NG - 1) % 2)

  plsc.subcore_barrier()

  @pl.loop(0, RPT // RB)
  def _rd(j):
    rs = s * RPT + j * RB
    pltpu.sync_copy(acc_sh.at[pl.ds(rs, RB)], bufs[0].at[pl.ds(0, RB)])
    pltpu.sync_copy(bufs[0].at[pl.ds(0, RB)], out_hbm.at[c, pl.ds(rs, RB)])


def _make_scatter(D, C, G, NG, NSLOT, RB):
  scratch = ([pltpu.VMEM((G * C,), jnp.int32) for _ in range(4)]
             + [pltpu.VMEM((C, D), jnp.float32) for _ in range(NSLOT)]
             + [pltpu.MemorySpace.VMEM_SHARED((NP, D), jnp.float32)]
             + [pltpu.SemaphoreType.DMA] * (2 + 2 * NSLOT))
  return pl.kernel(
      functools.partial(_scatter_body, D, C, G, NG, NSLOT, RB),
      out_type=[jax.ShapeDtypeStruct((NC, NP, D), jnp.float32)],
      mesh=_mesh,
      scratch_types=scratch,
      compiler_params=_sc_params,
  )


_scatter128 = _make_scatter(DF, 40, 25, 10, 5, 40)
_scatter16 = _make_scatter(DC, 80, 25, 5, 5, 80)


def _dinv(dp_ref):
  deg = jnp.sum(dp_ref[...], axis=1)
  return jnp.where(deg > 0, lax.rsqrt(deg), 0.0)


def _tc1_body(dp_ref, x_ref, w1_ref, y_ref):
  xs = (x_ref[...] * _dinv(dp_ref)[:, None]).astype(jnp.bfloat16)
  for r in range(NREL):
    y_ref[r] = jnp.dot(xs, w1_ref[r * DF:(r + 1) * DF, :],
                       preferred_element_type=jnp.float32)


def _tc2_body(dp_ref, acc_ref, b1_ref, w2_ref, y_ref):
  dinv = _dinv(dp_ref)
  h1 = b1_ref[...] + dinv[:, None] * (acc_ref[0] + acc_ref[1])
  h1s = (h1 * dinv[:, None]).astype(jnp.bfloat16)
  for r in range(NREL):
    y_ref[r] = jnp.dot(h1s, w2_ref[r * DF:(r + 1) * DF, :],
                       preferred_element_type=jnp.float32)


def _tc3_body(dp_ref, acc_ref, b2_ref, o_ref):
  dinv = _dinv(dp_ref)
  logits = b2_ref[...] + dinv[:, None] * (acc_ref[0] + acc_ref[1])[:, :8]
  m = jnp.max(logits, axis=1, keepdims=True)
  z = jnp.exp(logits - m)
  o_ref[...] = logits - m - jnp.log(jnp.sum(z, axis=1, keepdims=True))


def _tc1(dp, x, w1b):
  return pl.pallas_call(
      _tc1_body,
      grid=(NB,),
      in_specs=[pl.BlockSpec((BN, NW), lambda i: (i, 0)),
                pl.BlockSpec((BN, DF), lambda i: (i, 0)),
                pl.BlockSpec((NREL * DF, DF), lambda i: (0, 0))],
      out_specs=pl.BlockSpec((NREL, BN, DF), lambda i: (0, i, 0)),
      out_shape=jax.ShapeDtypeStruct((NREL, N, DF), jnp.float32),
  )(dp, x, w1b)


def _tc2(dp, acc, b1, w2pb):
  return pl.pallas_call(
      _tc2_body,
      grid=(NB,),
      in_specs=[pl.BlockSpec((BN, NW), lambda i: (i, 0)),
                pl.BlockSpec((NC, BN, DF), lambda i: (0, i, 0)),
                pl.BlockSpec((1, DF), lambda i: (0, 0)),
                pl.BlockSpec((NREL * DF, DC), lambda i: (0, 0))],
      out_specs=pl.BlockSpec((NREL, BN, DC), lambda i: (0, i, 0)),
      out_shape=jax.ShapeDtypeStruct((NREL, N, DC), jnp.float32),
  )(dp, acc, b1, w2pb)


def _tc3(dp, acc, b2):
  return pl.pallas_call(
      _tc3_body,
      grid=(NB,),
      in_specs=[pl.BlockSpec((BN, NW), lambda i: (i, 0)),
                pl.BlockSpec((NC, BN, DC), lambda i: (0, i, 0)),
                pl.BlockSpec((1, 8), lambda i: (0, 0))],
      out_specs=pl.BlockSpec((BN, 8), lambda i: (i, 0)),
      out_shape=jax.ShapeDtypeStruct((N, 8), jnp.float32),
  )(dp, acc, b2)


@jax.jit
def kernel(x, edge_index, edge_relation, W1, b1, W2, b2):
  degp, gidx, rowf = _deg_gidx(edge_index, edge_relation)
  dp = degp.T

  y1 = _tc1(dp, x, W1.astype(jnp.bfloat16))
  (acc1,) = _scatter128(y1.reshape(NREL * N, DF), gidx, rowf)

  w2pb = jnp.pad(W2, ((0, 0), (0, DC - W2.shape[1]))).astype(jnp.bfloat16)
  y2 = _tc2(dp, acc1, b1.reshape(1, DF), w2pb)
  (acc2,) = _scatter16(y2.reshape(NREL * N, DC), gidx, rowf)

  return _tc3(dp, acc2, b2.reshape(1, 8))


# speedup vs baseline: 57.2336x; 1.0638x over previous
import functools

import jax
import jax.numpy as jnp
from jax import lax
from jax.experimental import pallas as pl
from jax.experimental.pallas import tpu as pltpu
from jax.experimental.pallas import tpu_sc as plsc

N = 10000
E = 320000
NREL = 4
DF = 128
DC = 16
NC, NS, L = 2, 16, 16
NW = NC * NS
EW = E // NW
NP = 10240
RPT = NP // NS
BN = 2000
NB = N // BN

_mesh = plsc.VectorSubcoreMesh(core_axis_name="c", subcore_axis_name="s",
                               num_cores=NC, num_subcores=NS)
_sc_params = pltpu.CompilerParams(needs_layout_passes=False,
                                  use_tc_tiling_on_sc=False)


def _deg_gidx_body(ei_hbm, rel_hbm, degp_hbm, gidx_hbm, gidx2_hbm, rowf_hbm,
                   row_v, col_v, rel_v, gidx_v, gidx2_v, deg_v, sem):
  c = lax.axis_index("c")
  s = lax.axis_index("s")
  wid = s * NC + c
  base = wid * EW
  pltpu.async_copy(ei_hbm.at[0, pl.ds(base, EW)], row_v, sem)
  pltpu.async_copy(ei_hbm.at[1, pl.ds(base, EW)], col_v, sem)
  pltpu.async_copy(rel_hbm.at[pl.ds(base, EW)], rel_v, sem)

  zeros = jnp.zeros((L,), jnp.float32)

  @pl.loop(0, NP // L)
  def _zero(i):
    deg_v[pl.ds(i * L, L)] = zeros

  pltpu.make_async_copy(ei_hbm.at[0, pl.ds(base, EW)], row_v, sem).wait()
  pltpu.make_async_copy(ei_hbm.at[1, pl.ds(base, EW)], col_v, sem).wait()
  pltpu.make_async_copy(rel_hbm.at[pl.ds(base, EW)], rel_v, sem).wait()

  ones = jnp.ones((L,), jnp.float32)
  four = jnp.full((L,), 4, jnp.int32)

  @pl.loop(0, EW // L)
  def _edges(i):
    sl = pl.ds(i * L, L)
    gidx_v[sl] = rel_v[sl] * N + col_v[sl]
    gidx2_v[sl] = col_v[sl] * four + rel_v[sl]
    plsc.addupdate_scatter(deg_v, [row_v[sl]], ones)

  pltpu.sync_copy(deg_v, degp_hbm.at[wid])
  pltpu.sync_copy(gidx_v, gidx_hbm.at[pl.ds(base, EW)])
  pltpu.sync_copy(gidx2_v, gidx2_hbm.at[pl.ds(base, EW)])
  pltpu.sync_copy(row_v, rowf_hbm.at[pl.ds(base, EW)])


_deg_gidx = pl.kernel(
    _deg_gidx_body,
    out_type=[jax.ShapeDtypeStruct((NW, NP), jnp.float32),
              jax.ShapeDtypeStruct((E,), jnp.int32),
              jax.ShapeDtypeStruct((E,), jnp.int32),
              jax.ShapeDtypeStruct((E,), jnp.int32)],
    mesh=_mesh,
    scratch_types=[pltpu.VMEM((EW,), jnp.int32),
                   pltpu.VMEM((EW,), jnp.int32),
                   pltpu.VMEM((EW,), jnp.int32),
                   pltpu.VMEM((EW,), jnp.int32),
                   pltpu.VMEM((EW,), jnp.int32),
                   pltpu.VMEM((NP,), jnp.float32),
                   pltpu.SemaphoreType.DMA],
    compiler_params=_sc_params,
)


def _scatter_body(D, C, G, NG, NSLOT, RB, table_hbm, gidx_hbm, ridx_hbm,
                  out_hbm, *scr):
  ivs = scr[0:2]
  rvs = scr[2:4]
  bufs = scr[4:4 + NSLOT]
  acc_sh = scr[4 + NSLOT]
  isems = scr[5 + NSLOT:7 + NSLOT]
  gsems = scr[7 + NSLOT:7 + 2 * NSLOT]
  ssems = scr[7 + 2 * NSLOT:7 + 3 * NSLOT]
  c = lax.axis_index("c")
  s = lax.axis_index("s")
  wid = s * NC + c
  wbase = wid * EW
  GC = G * C

  zeros = jnp.zeros((L,), jnp.float32)

  @pl.loop(0, RB)
  def _zr(r):
    @pl.loop(0, D // L)
    def _zc(k):
      bufs[0][r, pl.ds(k * L, L)] = zeros

  @pl.loop(0, RPT // RB)
  def _zs(j):
    pltpu.sync_copy(bufs[0].at[pl.ds(0, RB)],
                    acc_sh.at[pl.ds(s * RPT + j * RB, RB)])

  plsc.subcore_barrier()

  def idx_load(g, p):
    pltpu.async_copy(gidx_hbm.at[pl.ds(wbase + g * GC, GC)], ivs[p],
                     isems[p])
    pltpu.async_copy(ridx_hbm.at[pl.ds(wbase + g * GC, GC)], rvs[p],
                     isems[p])

  def idx_wait(g, p):
    pltpu.make_async_copy(gidx_hbm.at[pl.ds(wbase + g * GC, GC)], ivs[p],
                          isems[p]).wait()
    pltpu.make_async_copy(ridx_hbm.at[pl.ds(wbase + g * GC, GC)], rvs[p],
                          isems[p]).wait()

  def gstart(j, k, p):
    pltpu.async_copy(table_hbm.at[ivs[p].at[pl.ds(j * C, C)]], bufs[k],
                     gsems[k])

  def gwait(j, k, p):
    pltpu.make_async_copy(table_hbm.at[ivs[p].at[pl.ds(j * C, C)]],
                          bufs[k], gsems[k]).wait()

  def sstart(j, k, p):
    pltpu.async_copy(bufs[k], acc_sh.at[rvs[p].at[pl.ds(j * C, C)]],
                     ssems[k], add=True)

  def sdrain(k, p):
    pltpu.make_async_copy(bufs[k], acc_sh.at[rvs[p].at[pl.ds(0, C)]],
                          ssems[k]).wait()

  idx_load(0, 0)
  for g in range(NG):
    p = g % 2
    idx_wait(g, p)
    for k in range(NSLOT):
      if g > 0:
        sdrain(k, p)
      gstart(k, k, p)
    if g + 1 < NG:
      idx_load(g + 1, (g + 1) % 2)

    @pl.loop(0, G // NSLOT)
    def _pipe(t):
      for k in range(NSLOT):
        j = t * NSLOT + k
        gwait(j, k, p)
        sstart(j, k, p)

        @pl.when(j + NSLOT < G)
        def _():
          sdrain(k, p)
          gstart(j + NSLOT, k, p)

  for k in range(NSLOT):
    sdrain(k, (NG - 1) % 2)

  plsc.subcore_barrier()

  @pl.loop(0, RPT // RB)
  def _rd(j):
    rs = s * RPT + j * RB
    pltpu.sync_copy(acc_sh.at[pl.ds(rs, RB)], bufs[0].at[pl.ds(0, RB)])
    pltpu.sync_copy(bufs[0].at[pl.ds(0, RB)], out_hbm.at[c, pl.ds(rs, RB)])


def _make_scatter(D, C, G, NG, NSLOT, RB):
  scratch = ([pltpu.VMEM((G * C,), jnp.int32) for _ in range(4)]
             + [pltpu.VMEM((C, D), jnp.float32) for _ in range(NSLOT)]
             + [pltpu.MemorySpace.VMEM_SHARED((NP, D), jnp.float32)]
             + [pltpu.SemaphoreType.DMA] * (2 + 2 * NSLOT))
  return pl.kernel(
      functools.partial(_scatter_body, D, C, G, NG, NSLOT, RB),
      out_type=[jax.ShapeDtypeStruct((NC, NP, D), jnp.float32)],
      mesh=_mesh,
      scratch_types=scratch,
      compiler_params=_sc_params,
  )


_scatter128 = _make_scatter(DF, 40, 25, 10, 5, 40)
_scatter16 = _make_scatter(DC, 80, 25, 5, 5, 80)


def _dinv(dp_ref):
  deg = jnp.sum(dp_ref[...], axis=1)
  return jnp.where(deg > 0, lax.rsqrt(deg), 0.0)


def _tc1_body(dp_ref, x_ref, w1_ref, y_ref):
  xs = (x_ref[...] * _dinv(dp_ref)[:, None]).astype(jnp.bfloat16)
  for r in range(NREL):
    y_ref[r] = jnp.dot(xs, w1_ref[r * DF:(r + 1) * DF, :],
                       preferred_element_type=jnp.float32)


def _tc2_body(dp_ref, acc_ref, b1_ref, w2_ref, y_ref):
  dinv = _dinv(dp_ref)
  h1 = b1_ref[...] + dinv[:, None] * (acc_ref[0] + acc_ref[1])
  h1s = (h1 * dinv[:, None]).astype(jnp.bfloat16)
  ys = [jnp.dot(h1s, w2_ref[r * DF:(r + 1) * DF, :],
                preferred_element_type=jnp.float32) for r in range(NREL)]
  y_ref[...] = jnp.concatenate(ys, axis=1)


def _tc3_body(dp_ref, acc_ref, b2_ref, o_ref):
  dinv = _dinv(dp_ref)
  logits = b2_ref[...] + dinv[:, None] * (acc_ref[0] + acc_ref[1])[:, :8]
  m = jnp.max(logits, axis=1, keepdims=True)
  z = jnp.exp(logits - m)
  o_ref[...] = logits - m - jnp.log(jnp.sum(z, axis=1, keepdims=True))


def _tc1(dp, x, w1b):
  return pl.pallas_call(
      _tc1_body,
      grid=(NB,),
      in_specs=[pl.BlockSpec((BN, NW), lambda i: (i, 0)),
                pl.BlockSpec((BN, DF), lambda i: (i, 0)),
                pl.BlockSpec((NREL * DF, DF), lambda i: (0, 0))],
      out_specs=pl.BlockSpec((NREL, BN, DF), lambda i: (0, i, 0)),
      out_shape=jax.ShapeDtypeStruct((NREL, N, DF), jnp.float32),
  )(dp, x, w1b)


def _tc2(dp, acc, b1, w2pb):
  return pl.pallas_call(
      _tc2_body,
      grid=(NB,),
      in_specs=[pl.BlockSpec((BN, NW), lambda i: (i, 0)),
                pl.BlockSpec((NC, BN, DF), lambda i: (0, i, 0)),
                pl.BlockSpec((1, DF), lambda i: (0, 0)),
                pl.BlockSpec((NREL * DF, DC), lambda i: (0, 0))],
      out_specs=pl.BlockSpec((BN, NREL * DC), lambda i: (i, 0)),
      out_shape=jax.ShapeDtypeStruct((N, NREL * DC), jnp.float32),
  )(dp, acc, b1, w2pb)


def _tc3(dp, acc, b2):
  return pl.pallas_call(
      _tc3_body,
      grid=(NB,),
      in_specs=[pl.BlockSpec((BN, NW), lambda i: (i, 0)),
                pl.BlockSpec((NC, BN, DC), lambda i: (0, i, 0)),
                pl.BlockSpec((1, 8), lambda i: (0, 0))],
      out_specs=pl.BlockSpec((BN, 8), lambda i: (i, 0)),
      out_shape=jax.ShapeDtypeStruct((N, 8), jnp.float32),
  )(dp, acc, b2)


@jax.jit
def kernel(x, edge_index, edge_relation, W1, b1, W2, b2):
  degp, gidx, gidx2, rowf = _deg_gidx(edge_index, edge_relation)
  dp = degp.T

  y1 = _tc1(dp, x, W1.astype(jnp.bfloat16))
  (acc1,) = _scatter128(y1.reshape(NREL * N, DF), gidx, rowf)

  w2pb = jnp.pad(W2, ((0, 0), (0, DC - W2.shape[1]))).astype(jnp.bfloat16)
  y2 = _tc2(dp, acc1, b1.reshape(1, DF), w2pb)
  (acc2,) = _scatter16(y2.reshape(NREL * N, DC), gidx2, rowf)

  return _tc3(dp, acc2, b2.reshape(1, 8))
